# baseline (speedup 1.0000x reference)
import functools

import jax
import jax.numpy as jnp
from jax import lax
from jax.experimental import pallas as pl
from jax.experimental.pallas import tpu as pltpu
from jax.experimental.pallas import tpu_sc as plsc

N = 10000
N_PAD = 10240
E = 320000
CHUNK = 128
TOTAL_CHUNKS = 2560
E_PAD = TOTAL_CHUNKS * CHUNK
D_IN, D_HID, D_OUT = 128, 64, 32
NC, NS = 2, 16
ROWS_PER_TILE = N_PAD // NS
DEG_W = 16
BN = 2560

K0_D, K1_D = 96, 64
C1_BASE_D = NS * K0_D
K0_MAX = 148


def _sc_mesh():
    return plsc.VectorSubcoreMesh(core_axis_name="c", subcore_axis_name="s")


@functools.partial(
    pl.kernel,
    mesh=_sc_mesh(),
    compiler_params=pltpu.CompilerParams(use_tc_tiling_on_sc=False),
    out_type=jax.ShapeDtypeStruct((NC, N_PAD, DEG_W), jnp.float32),
    scratch_types=[
        pltpu.VMEM((K0_D, CHUNK), jnp.int32),
        pltpu.VMEM((CHUNK, DEG_W), jnp.float32),
        pltpu.VMEM((CHUNK, DEG_W), jnp.float32),
        pltpu.VMEM_SHARED((N_PAD, DEG_W), jnp.float32),
        pltpu.SemaphoreType.DMA,
    ],
)
def _deg_kernel(dst_hbm, ones_hbm, out_hbm, dst_v, ones_v, zrow_v, accum, sem):
    c = lax.axis_index("c")
    s = lax.axis_index("s")
    base = s * ROWS_PER_TILE

    def zstore(i, carry):
        zrow_v[i, pl.ds(0, DEG_W)] = jnp.zeros((DEG_W,), jnp.float32)
        return carry

    lax.fori_loop(0, CHUNK, zstore, 0)
    for i in range(ROWS_PER_TILE // CHUNK):
        pltpu.sync_copy(zrow_v, accum.at[pl.ds(base + i * CHUNK, CHUNK)])

    pltpu.sync_copy(ones_hbm, ones_v)

    def stage_idx(cbase, k):
        pltpu.sync_copy(dst_hbm.at[pl.ds(cbase, k)], dst_v.at[pl.ds(0, k)])

    pl.when(c == 0)(lambda: stage_idx(s * K0_D, K0_D))
    pl.when(c == 1)(lambda: stage_idx(C1_BASE_D + s * K1_D, K1_D))
    plsc.subcore_barrier()

    def run(k):
        def body(j, carry):
            pltpu.async_copy(ones_v, accum.at[dst_v.at[j]], sem, add=True)
            return carry

        lax.fori_loop(0, k, body, 0)

        def drain(j, carry):
            pltpu.make_async_copy(ones_v, accum.at[dst_v.at[j]], sem).wait()
            return carry

        lax.fori_loop(0, k, drain, 0)

    pl.when(c == 0)(lambda: run(K0_D))
    pl.when(c == 1)(lambda: run(K1_D))
    plsc.subcore_barrier()
    for i in range(ROWS_PER_TILE // CHUNK):
        pltpu.sync_copy(accum.at[pl.ds(base + i * CHUNK, CHUNK)], zrow_v)
        pltpu.sync_copy(zrow_v,
                        out_hbm.at[c].at[pl.ds(base + i * CHUNK, CHUNK)])


NBUF = 4
K_EDGE = TOTAL_CHUNKS // NS


def _make_edge_kernel(d):
    @functools.partial(
        pl.kernel,
        mesh=_sc_mesh(),
        compiler_params=pltpu.CompilerParams(use_tc_tiling_on_sc=False),
        out_type=jax.ShapeDtypeStruct((N_PAD, d), jnp.float32),
        scratch_types=[
            pltpu.VMEM((K_EDGE, CHUNK), jnp.int32),
            pltpu.VMEM((K_EDGE, CHUNK), jnp.int32),
            pltpu.VMEM((NBUF, CHUNK, d), jnp.float32),
            pltpu.VMEM_SHARED((N_PAD, d), jnp.float32),
            pltpu.SemaphoreType.DMA((NBUF,)),
            pltpu.SemaphoreType.DMA((NBUF,)),
        ],
    )
    def edge_kernel(y_hbm, src_hbm, dst_hbm, out_hbm,
                    src_v, dst_v, rows_v, accum, gsems, ssems):
        c = lax.axis_index("c")
        s = lax.axis_index("s")

        @pl.when(c == 0)
        def _():
            base = s * ROWS_PER_TILE
            k = K_EDGE
            ngrp = k // NBUF

            def zstore(i, carry):
                for j in range(d // 16):
                    rows_v[0, i, pl.ds(j * 16, 16)] = jnp.zeros(
                        (16,), jnp.float32)
                return carry

            lax.fori_loop(0, CHUNK, zstore, 0)
            for i in range(ROWS_PER_TILE // CHUNK):
                pltpu.sync_copy(rows_v.at[0],
                                accum.at[pl.ds(base + i * CHUNK, CHUNK)])

            pltpu.sync_copy(src_hbm.at[pl.ds(s * k, k)], src_v)
            pltpu.sync_copy(dst_hbm.at[pl.ds(s * k, k)], dst_v)
            plsc.subcore_barrier()

            def wait_gather(t, b):
                pltpu.make_async_copy(y_hbm.at[src_v.at[t * NBUF + b]],
                                      rows_v.at[b], gsems.at[b]).wait()

            def wait_scatter(t, b):
                pltpu.make_async_copy(rows_v.at[b],
                                      accum.at[dst_v.at[t * NBUF + b]],
                                      ssems.at[b]).wait()

            for b in range(NBUF):
                pltpu.async_copy(y_hbm.at[src_v.at[b]], rows_v.at[b],
                                 gsems.at[b])

            def group(t, carry):
                for b in range(NBUF):
                    wait_gather(t, b)
                    pltpu.async_copy(rows_v.at[b],
                                     accum.at[dst_v.at[t * NBUF + b]],
                                     ssems.at[b], add=True)

                @pl.when(t < ngrp - 1)
                def _():
                    for b in range(NBUF):
                        wait_scatter(t, b)
                        pltpu.async_copy(y_hbm.at[src_v.at[(t + 1) * NBUF + b]],
                                         rows_v.at[b], gsems.at[b])

                return carry

            lax.fori_loop(0, ngrp, group, 0)
            for b in range(NBUF):
                wait_scatter(ngrp - 1, b)
            plsc.subcore_barrier()

            prev = [None, None]
            for i in range(ROWS_PER_TILE // CHUNK):
                b = i % 2
                if prev[b] is not None:
                    prev[b].wait()
                pltpu.async_copy(accum.at[pl.ds(base + i * CHUNK, CHUNK)],
                                 rows_v.at[b], gsems.at[b]).wait()
                prev[b] = pltpu.async_copy(
                    rows_v.at[b],
                    out_hbm.at[pl.ds(base + i * CHUNK, CHUNK)],
                    ssems.at[b])
            for p in prev:
                p.wait()

    return edge_kernel


_edge_kernel_h = _make_edge_kernel(D_HID)
_edge_kernel_o = _make_edge_kernel(D_OUT)


def _dinv_block(degp):
    deg = degp[0, :, 0:1] + degp[1, :, 0:1] + 1.0
    return lax.rsqrt(deg)


def _tc1_body(x_ref, w_ref, degp_ref, y_ref):
    dinv = _dinv_block(degp_ref[...])
    xw = jnp.dot(x_ref[...], w_ref[...], preferred_element_type=jnp.float32)
    y_ref[...] = xw * dinv


_tc1 = pl.pallas_call(
    _tc1_body,
    grid=(N_PAD // BN,),
    in_specs=[
        pl.BlockSpec((BN, D_IN), lambda i: (i, 0)),
        pl.BlockSpec((D_IN, D_HID), lambda i: (0, 0)),
        pl.BlockSpec((NC, BN, DEG_W), lambda i: (0, i, 0)),
    ],
    out_specs=pl.BlockSpec((BN, D_HID), lambda i: (i, 0)),
    out_shape=jax.ShapeDtypeStruct((N_PAD, D_HID), jnp.float32),
)


def _tc2_body(part_ref, y1_ref, degp_ref, b1_ref, w2_ref, y2_ref):
    dinv = _dinv_block(degp_ref[...])
    pre = (part_ref[...] + y1_ref[...]) * dinv + b1_ref[...]
    h = jnp.maximum(pre, 0.0)
    y2_ref[...] = jnp.dot(h, w2_ref[...],
                          preferred_element_type=jnp.float32) * dinv


_tc2 = pl.pallas_call(
    _tc2_body,
    grid=(N_PAD // BN,),
    in_specs=[
        pl.BlockSpec((BN, D_HID), lambda i: (i, 0)),
        pl.BlockSpec((BN, D_HID), lambda i: (i, 0)),
        pl.BlockSpec((NC, BN, DEG_W), lambda i: (0, i, 0)),
        pl.BlockSpec((1, D_HID), lambda i: (0, 0)),
        pl.BlockSpec((D_HID, D_OUT), lambda i: (0, 0)),
    ],
    out_specs=pl.BlockSpec((BN, D_OUT), lambda i: (i, 0)),
    out_shape=jax.ShapeDtypeStruct((N_PAD, D_OUT), jnp.float32),
)


def _tc3_body(part_ref, y2_ref, degp_ref, b2_ref, out_ref):
    dinv = _dinv_block(degp_ref[...])
    out_ref[...] = (part_ref[...] + y2_ref[...]) * dinv + b2_ref[...]


_tc3 = pl.pallas_call(
    _tc3_body,
    grid=(N_PAD // BN,),
    in_specs=[
        pl.BlockSpec((BN, D_OUT), lambda i: (i, 0)),
        pl.BlockSpec((BN, D_OUT), lambda i: (i, 0)),
        pl.BlockSpec((NC, BN, DEG_W), lambda i: (0, i, 0)),
        pl.BlockSpec((1, D_OUT), lambda i: (0, 0)),
    ],
    out_specs=pl.BlockSpec((BN, D_OUT), lambda i: (i, 0)),
    out_shape=jax.ShapeDtypeStruct((N_PAD, D_OUT), jnp.float32),
)


def kernel(x, edge_index, W1, b1, W2, b2):
    pad_e = E_PAD - E
    src3 = jnp.concatenate(
        [edge_index[0], jnp.full((pad_e,), N, jnp.int32)]
    ).reshape(TOTAL_CHUNKS, CHUNK)
    dst3 = jnp.concatenate(
        [edge_index[1], jnp.full((pad_e,), N, jnp.int32)]
    ).reshape(TOTAL_CHUNKS, CHUNK)
    x_pad = jnp.zeros((N_PAD, D_IN), jnp.float32).at[:N].set(x)

    ones_w = jnp.ones((CHUNK, DEG_W), jnp.float32)

    degp = _deg_kernel(dst3, ones_w)
    y1 = _tc1(x_pad, W1, degp)
    part1 = _edge_kernel_h(y1, src3, dst3)
    y2 = _tc2(part1, y1, degp, b1.reshape(1, D_HID), W2)
    part2 = _edge_kernel_o(y2, src3, dst3)
    out = _tc3(part2, y2, degp, b2.reshape(1, D_OUT))
    return out[:N]


# speedup vs baseline: 1.1675x; 1.1675x over previous
import functools

import jax
import jax.numpy as jnp
from jax import lax
from jax.experimental import pallas as pl
from jax.experimental.pallas import tpu as pltpu
from jax.experimental.pallas import tpu_sc as plsc

N = 10000
N_PAD = 10240
E = 320000
CHUNK = 128
TOTAL_CHUNKS = 2560
E_PAD = TOTAL_CHUNKS * CHUNK
D_IN, D_HID, D_OUT = 128, 64, 32
NC, NS = 2, 16
ROWS_PER_TILE = N_PAD // NS
DEG_W = 16
BN = 2560

K0_D, K1_D = 96, 64
C1_BASE_D = NS * K0_D
K0_MAX = 136


def _sc_mesh():
    return plsc.VectorSubcoreMesh(core_axis_name="c", subcore_axis_name="s")


@functools.partial(
    pl.kernel,
    mesh=_sc_mesh(),
    compiler_params=pltpu.CompilerParams(use_tc_tiling_on_sc=False),
    out_type=jax.ShapeDtypeStruct((NC, N_PAD, DEG_W), jnp.float32),
    scratch_types=[
        pltpu.VMEM((K0_D, CHUNK), jnp.int32),
        pltpu.VMEM((CHUNK, DEG_W), jnp.float32),
        pltpu.VMEM((CHUNK, DEG_W), jnp.float32),
        pltpu.VMEM_SHARED((N_PAD, DEG_W), jnp.float32),
        pltpu.SemaphoreType.DMA,
    ],
)
def _deg_kernel(dst_hbm, ones_hbm, out_hbm, dst_v, ones_v, zrow_v, accum, sem):
    c = lax.axis_index("c")
    s = lax.axis_index("s")
    base = s * ROWS_PER_TILE

    def zstore(i, carry):
        zrow_v[i, pl.ds(0, DEG_W)] = jnp.zeros((DEG_W,), jnp.float32)
        return carry

    lax.fori_loop(0, CHUNK, zstore, 0)
    for i in range(ROWS_PER_TILE // CHUNK):
        pltpu.sync_copy(zrow_v, accum.at[pl.ds(base + i * CHUNK, CHUNK)])

    pltpu.sync_copy(ones_hbm, ones_v)

    def stage_idx(cbase, k):
        pltpu.sync_copy(dst_hbm.at[pl.ds(cbase, k)], dst_v.at[pl.ds(0, k)])

    pl.when(c == 0)(lambda: stage_idx(s * K0_D, K0_D))
    pl.when(c == 1)(lambda: stage_idx(C1_BASE_D + s * K1_D, K1_D))
    plsc.subcore_barrier()

    def run(k):
        def body(j, carry):
            pltpu.async_copy(ones_v, accum.at[dst_v.at[j]], sem, add=True)
            return carry

        lax.fori_loop(0, k, body, 0)

        def drain(j, carry):
            pltpu.make_async_copy(ones_v, accum.at[dst_v.at[j]], sem).wait()
            return carry

        lax.fori_loop(0, k, drain, 0)

    pl.when(c == 0)(lambda: run(K0_D))
    pl.when(c == 1)(lambda: run(K1_D))
    plsc.subcore_barrier()
    for i in range(ROWS_PER_TILE // CHUNK):
        pltpu.sync_copy(accum.at[pl.ds(base + i * CHUNK, CHUNK)], zrow_v)
        pltpu.sync_copy(zrow_v,
                        out_hbm.at[c].at[pl.ds(base + i * CHUNK, CHUNK)])


NBUF = 4


def _make_edge_kernel(d, k0, k1):
    c1_base = NS * k0

    @functools.partial(
        pl.kernel,
        mesh=_sc_mesh(),
        compiler_params=pltpu.CompilerParams(use_tc_tiling_on_sc=False),
        out_type=jax.ShapeDtypeStruct((NC, N_PAD, d), jnp.float32),
        scratch_types=[
            pltpu.VMEM((K0_MAX, CHUNK), jnp.int32),
            pltpu.VMEM((K0_MAX, CHUNK), jnp.int32),
            pltpu.VMEM((NBUF, CHUNK, d), jnp.float32),
            pltpu.VMEM_SHARED((N_PAD, d), jnp.float32),
            pltpu.SemaphoreType.DMA((NBUF,)),
            pltpu.SemaphoreType.DMA((NBUF,)),
        ],
    )
    def edge_kernel(y_hbm, src_hbm, dst_hbm, out_hbm,
                    src_v, dst_v, rows_v, accum, gsems, ssems):
        c = lax.axis_index("c")
        s = lax.axis_index("s")
        base = s * ROWS_PER_TILE

        def zstore(i, carry):
            for j in range(d // 16):
                rows_v[0, i, pl.ds(j * 16, 16)] = jnp.zeros((16,), jnp.float32)
            return carry

        lax.fori_loop(0, CHUNK, zstore, 0)
        for i in range(ROWS_PER_TILE // CHUNK):
            pltpu.sync_copy(rows_v.at[0],
                            accum.at[pl.ds(base + i * CHUNK, CHUNK)])

        def stage_idx(cbase, k):
            pltpu.sync_copy(src_hbm.at[pl.ds(cbase, k)], src_v.at[pl.ds(0, k)])
            pltpu.sync_copy(dst_hbm.at[pl.ds(cbase, k)], dst_v.at[pl.ds(0, k)])

        pl.when(c == 0)(lambda: stage_idx(s * k0, k0))
        pl.when(c == 1)(lambda: stage_idx(c1_base + s * k1, k1))
        plsc.subcore_barrier()

        def wait_gather(t, b):
            pltpu.make_async_copy(y_hbm.at[src_v.at[t * NBUF + b]],
                                  rows_v.at[b], gsems.at[b]).wait()

        def wait_scatter(t, b):
            pltpu.make_async_copy(rows_v.at[b],
                                  accum.at[dst_v.at[t * NBUF + b]],
                                  ssems.at[b]).wait()

        def run(k):
            ngrp = k // NBUF
            for b in range(NBUF):
                pltpu.async_copy(y_hbm.at[src_v.at[b]], rows_v.at[b],
                                 gsems.at[b])

            def group(t, carry):
                for b in range(NBUF):
                    wait_gather(t, b)
                    pltpu.async_copy(rows_v.at[b],
                                     accum.at[dst_v.at[t * NBUF + b]],
                                     ssems.at[b], add=True)

                @pl.when(t < ngrp - 1)
                def _():
                    for b in range(NBUF):
                        wait_scatter(t, b)
                        pltpu.async_copy(y_hbm.at[src_v.at[(t + 1) * NBUF + b]],
                                         rows_v.at[b], gsems.at[b])

                return carry

            lax.fori_loop(0, ngrp, group, 0)
            for b in range(NBUF):
                wait_scatter(ngrp - 1, b)

        pl.when(c == 0)(lambda: run(k0))
        pl.when(c == 1)(lambda: run(k1))
        plsc.subcore_barrier()

        descs = [None] * NBUF
        for i in range(ROWS_PER_TILE // CHUNK):
            b = i % NBUF
            if descs[b] is not None:
                descs[b].wait()
            pltpu.async_copy(accum.at[pl.ds(base + i * CHUNK, CHUNK)],
                             rows_v.at[b], gsems.at[b]).wait()
            descs[b] = pltpu.async_copy(
                rows_v.at[b],
                out_hbm.at[c].at[pl.ds(base + i * CHUNK, CHUNK)],
                ssems.at[b])
        for p in descs:
            if p is not None:
                p.wait()

    return edge_kernel


_edge_kernel_h = _make_edge_kernel(D_HID, 136, 24)
_edge_kernel_o = _make_edge_kernel(D_OUT, 136, 24)


def _dinv_block(degp):
    deg = degp[0, :, 0:1] + degp[1, :, 0:1] + 1.0
    return lax.rsqrt(deg)


def _tc1_body(x_ref, w_ref, degp_ref, y_ref):
    dinv = _dinv_block(degp_ref[...])
    xw = jnp.dot(x_ref[...], w_ref[...], preferred_element_type=jnp.float32)
    y_ref[...] = xw * dinv


_tc1 = pl.pallas_call(
    _tc1_body,
    grid=(N_PAD // BN,),
    in_specs=[
        pl.BlockSpec((BN, D_IN), lambda i: (i, 0)),
        pl.BlockSpec((D_IN, D_HID), lambda i: (0, 0)),
        pl.BlockSpec((NC, BN, DEG_W), lambda i: (0, i, 0)),
    ],
    out_specs=pl.BlockSpec((BN, D_HID), lambda i: (i, 0)),
    out_shape=jax.ShapeDtypeStruct((N_PAD, D_HID), jnp.float32),
)


def _tc2_body(part_ref, y1_ref, degp_ref, b1_ref, w2_ref, y2_ref):
    dinv = _dinv_block(degp_ref[...])
    p = part_ref[...]
    pre = (p[0] + p[1] + y1_ref[...]) * dinv + b1_ref[...]
    h = jnp.maximum(pre, 0.0)
    y2_ref[...] = jnp.dot(h, w2_ref[...],
                          preferred_element_type=jnp.float32) * dinv


_tc2 = pl.pallas_call(
    _tc2_body,
    grid=(N_PAD // BN,),
    in_specs=[
        pl.BlockSpec((NC, BN, D_HID), lambda i: (0, i, 0)),
        pl.BlockSpec((BN, D_HID), lambda i: (i, 0)),
        pl.BlockSpec((NC, BN, DEG_W), lambda i: (0, i, 0)),
        pl.BlockSpec((1, D_HID), lambda i: (0, 0)),
        pl.BlockSpec((D_HID, D_OUT), lambda i: (0, 0)),
    ],
    out_specs=pl.BlockSpec((BN, D_OUT), lambda i: (i, 0)),
    out_shape=jax.ShapeDtypeStruct((N_PAD, D_OUT), jnp.float32),
)


def _tc3_body(part_ref, y2_ref, degp_ref, b2_ref, out_ref):
    dinv = _dinv_block(degp_ref[...])
    p = part_ref[...]
    out_ref[...] = (p[0] + p[1] + y2_ref[...]) * dinv + b2_ref[...]


_tc3 = pl.pallas_call(
    _tc3_body,
    grid=(N_PAD // BN,),
    in_specs=[
        pl.BlockSpec((NC, BN, D_OUT), lambda i: (0, i, 0)),
        pl.BlockSpec((BN, D_OUT), lambda i: (i, 0)),
        pl.BlockSpec((NC, BN, DEG_W), lambda i: (0, i, 0)),
        pl.BlockSpec((1, D_OUT), lambda i: (0, 0)),
    ],
    out_specs=pl.BlockSpec((BN, D_OUT), lambda i: (i, 0)),
    out_shape=jax.ShapeDtypeStruct((N_PAD, D_OUT), jnp.float32),
)


def kernel(x, edge_index, W1, b1, W2, b2):
    pad_e = E_PAD - E
    src3 = jnp.concatenate(
        [edge_index[0], jnp.full((pad_e,), N, jnp.int32)]
    ).reshape(TOTAL_CHUNKS, CHUNK)
    dst3 = jnp.concatenate(
        [edge_index[1], jnp.full((pad_e,), N, jnp.int32)]
    ).reshape(TOTAL_CHUNKS, CHUNK)
    x_pad = jnp.zeros((N_PAD, D_IN), jnp.float32).at[:N].set(x)

    ones_w = jnp.ones((CHUNK, DEG_W), jnp.float32)

    degp = _deg_kernel(dst3, ones_w)
    y1 = _tc1(x_pad, W1, degp)
    part1 = _edge_kernel_h(y1, src3, dst3)
    y2 = _tc2(part1, y1, degp, b1.reshape(1, D_HID), W2)
    part2 = _edge_kernel_o(y2, src3, dst3)
    out = _tc3(part2, y2, degp, b2.reshape(1, D_OUT))
    return out[:N]


# speedup vs baseline: 1.3589x; 1.1640x over previous
import functools

import jax
import jax.numpy as jnp
from jax import lax
from jax.experimental import pallas as pl
from jax.experimental.pallas import tpu as pltpu
from jax.experimental.pallas import tpu_sc as plsc

N = 10000
N_PAD = 10240
E = 320000
CHUNK = 128
TOTAL_CHUNKS = 2560
E_PAD = TOTAL_CHUNKS * CHUNK
D_IN, D_HID, D_OUT = 128, 64, 32
NC, NS = 2, 16
ROWS_PER_TILE = N_PAD // NS
DEG_W = 16
BN = 2560

K0_D, K1_D = 96, 64
C1_BASE_D = NS * K0_D
K0_MAX = 148


def _sc_mesh():
    return plsc.VectorSubcoreMesh(core_axis_name="c", subcore_axis_name="s")


@functools.partial(
    pl.kernel,
    mesh=_sc_mesh(),
    compiler_params=pltpu.CompilerParams(use_tc_tiling_on_sc=False),
    out_type=jax.ShapeDtypeStruct((NC, N_PAD, DEG_W), jnp.float32),
    scratch_types=[
        pltpu.VMEM((K0_D, CHUNK), jnp.int32),
        pltpu.VMEM((CHUNK, DEG_W), jnp.float32),
        pltpu.VMEM((CHUNK, DEG_W), jnp.float32),
        pltpu.VMEM_SHARED((N_PAD, DEG_W), jnp.float32),
        pltpu.SemaphoreType.DMA,
    ],
)
def _deg_kernel(dst_hbm, ones_hbm, out_hbm, dst_v, ones_v, zrow_v, accum, sem):
    c = lax.axis_index("c")
    s = lax.axis_index("s")
    base = s * ROWS_PER_TILE

    def zstore(i, carry):
        zrow_v[i, pl.ds(0, DEG_W)] = jnp.zeros((DEG_W,), jnp.float32)
        return carry

    lax.fori_loop(0, CHUNK, zstore, 0)
    for i in range(ROWS_PER_TILE // CHUNK):
        pltpu.sync_copy(zrow_v, accum.at[pl.ds(base + i * CHUNK, CHUNK)])

    pltpu.sync_copy(ones_hbm, ones_v)

    def stage_idx(cbase, k):
        pltpu.sync_copy(dst_hbm.at[pl.ds(cbase, k)], dst_v.at[pl.ds(0, k)])

    pl.when(c == 0)(lambda: stage_idx(s * K0_D, K0_D))
    pl.when(c == 1)(lambda: stage_idx(C1_BASE_D + s * K1_D, K1_D))
    plsc.subcore_barrier()

    def run(k):
        def body(j, carry):
            pltpu.async_copy(ones_v, accum.at[dst_v.at[j]], sem, add=True)
            return carry

        lax.fori_loop(0, k, body, 0)

        def drain(j, carry):
            pltpu.make_async_copy(ones_v, accum.at[dst_v.at[j]], sem).wait()
            return carry

        lax.fori_loop(0, k, drain, 0)

    pl.when(c == 0)(lambda: run(K0_D))
    pl.when(c == 1)(lambda: run(K1_D))
    plsc.subcore_barrier()
    for i in range(ROWS_PER_TILE // CHUNK):
        pltpu.sync_copy(accum.at[pl.ds(base + i * CHUNK, CHUNK)], zrow_v)
        pltpu.sync_copy(zrow_v,
                        out_hbm.at[c].at[pl.ds(base + i * CHUNK, CHUNK)])


NBUF = 4


def _make_edge_kernel(d, k0, k1):
    c1_base = NS * k0

    @functools.partial(
        pl.kernel,
        mesh=_sc_mesh(),
        compiler_params=pltpu.CompilerParams(use_tc_tiling_on_sc=False),
        out_type=jax.ShapeDtypeStruct((NC, N_PAD, d), jnp.float32),
        scratch_types=[
            pltpu.VMEM((K0_MAX, CHUNK), jnp.int32),
            pltpu.VMEM((K0_MAX, CHUNK), jnp.int32),
            pltpu.VMEM((NBUF, CHUNK, d), jnp.float32),
            pltpu.VMEM_SHARED((N_PAD, d), jnp.float32),
            pltpu.SemaphoreType.DMA((NBUF,)),
            pltpu.SemaphoreType.DMA((NBUF,)),
        ],
    )
    def edge_kernel(y_hbm, src_hbm, dst_hbm, out_hbm,
                    src_v, dst_v, rows_v, accum, gsems, ssems):
        c = lax.axis_index("c")
        s = lax.axis_index("s")
        base = s * ROWS_PER_TILE

        def zstore(i, carry):
            for j in range(d // 16):
                rows_v[0, i, pl.ds(j * 16, 16)] = jnp.zeros((16,), jnp.float32)
            return carry

        lax.fori_loop(0, CHUNK, zstore, 0)
        for i in range(ROWS_PER_TILE // CHUNK):
            pltpu.sync_copy(rows_v.at[0],
                            accum.at[pl.ds(base + i * CHUNK, CHUNK)])

        def stage_idx(cbase, k):
            pltpu.sync_copy(src_hbm.at[pl.ds(cbase, k)], src_v.at[pl.ds(0, k)])
            pltpu.sync_copy(dst_hbm.at[pl.ds(cbase, k)], dst_v.at[pl.ds(0, k)])

        pl.when(c == 0)(lambda: stage_idx(s * k0, k0))
        pl.when(c == 1)(lambda: stage_idx(c1_base + s * k1, k1))
        plsc.subcore_barrier()

        def wait_gather(t, b):
            pltpu.make_async_copy(y_hbm.at[src_v.at[t * NBUF + b]],
                                  rows_v.at[b], gsems.at[b]).wait()

        def wait_scatter(t, b):
            pltpu.make_async_copy(rows_v.at[b],
                                  accum.at[dst_v.at[t * NBUF + b]],
                                  ssems.at[b]).wait()

        def run(k):
            ngrp = k // NBUF
            for b in range(NBUF):
                pltpu.async_copy(y_hbm.at[src_v.at[b]], rows_v.at[b],
                                 gsems.at[b])

            def group(t, carry):
                for b in range(NBUF):
                    wait_gather(t, b)
                    pltpu.async_copy(rows_v.at[b],
                                     accum.at[dst_v.at[t * NBUF + b]],
                                     ssems.at[b], add=True)

                @pl.when(t < ngrp - 1)
                def _():
                    for b in range(NBUF):
                        wait_scatter(t, b)
                        pltpu.async_copy(y_hbm.at[src_v.at[(t + 1) * NBUF + b]],
                                         rows_v.at[b], gsems.at[b])

                return carry

            lax.fori_loop(0, ngrp, group, 0)
            for b in range(NBUF):
                wait_scatter(ngrp - 1, b)

        pl.when(c == 0)(lambda: run(k0))
        pl.when(c == 1)(lambda: run(k1))
        plsc.subcore_barrier()

        descs = [None] * NBUF
        for i in range(ROWS_PER_TILE // CHUNK):
            b = i % NBUF
            if descs[b] is not None:
                descs[b].wait()
            pltpu.async_copy(accum.at[pl.ds(base + i * CHUNK, CHUNK)],
                             rows_v.at[b], gsems.at[b]).wait()
            descs[b] = pltpu.async_copy(
                rows_v.at[b],
                out_hbm.at[c].at[pl.ds(base + i * CHUNK, CHUNK)],
                ssems.at[b])
        for p in descs:
            if p is not None:
                p.wait()

    return edge_kernel


_edge_kernel_h = _make_edge_kernel(D_HID, 148, 12)
_edge_kernel_o = _make_edge_kernel(D_OUT, 144, 16)


def _dinv_block(degp):
    deg = degp[0, :, 0:1] + degp[1, :, 0:1] + 1.0
    return lax.rsqrt(deg)


def _tc1_body(x_ref, w_ref, degp_ref, y_ref):
    dinv = _dinv_block(degp_ref[...])
    xw = jnp.dot(x_ref[...], w_ref[...], preferred_element_type=jnp.float32)
    y_ref[...] = xw * dinv


_tc1 = pl.pallas_call(
    _tc1_body,
    grid=(N_PAD // BN,),
    in_specs=[
        pl.BlockSpec((BN, D_IN), lambda i: (i, 0)),
        pl.BlockSpec((D_IN, D_HID), lambda i: (0, 0)),
        pl.BlockSpec((NC, BN, DEG_W), lambda i: (0, i, 0)),
    ],
    out_specs=pl.BlockSpec((BN, D_HID), lambda i: (i, 0)),
    out_shape=jax.ShapeDtypeStruct((N_PAD, D_HID), jnp.float32),
)


def _tc2_body(part_ref, y1_ref, degp_ref, b1_ref, w2_ref, y2_ref):
    dinv = _dinv_block(degp_ref[...])
    p = part_ref[...]
    pre = (p[0] + p[1] + y1_ref[...]) * dinv + b1_ref[...]
    h = jnp.maximum(pre, 0.0)
    y2_ref[...] = jnp.dot(h, w2_ref[...],
                          preferred_element_type=jnp.float32) * dinv


_tc2 = pl.pallas_call(
    _tc2_body,
    grid=(N_PAD // BN,),
    in_specs=[
        pl.BlockSpec((NC, BN, D_HID), lambda i: (0, i, 0)),
        pl.BlockSpec((BN, D_HID), lambda i: (i, 0)),
        pl.BlockSpec((NC, BN, DEG_W), lambda i: (0, i, 0)),
        pl.BlockSpec((1, D_HID), lambda i: (0, 0)),
        pl.BlockSpec((D_HID, D_OUT), lambda i: (0, 0)),
    ],
    out_specs=pl.BlockSpec((BN, D_OUT), lambda i: (i, 0)),
    out_shape=jax.ShapeDtypeStruct((N_PAD, D_OUT), jnp.float32),
)


def _tc3_body(part_ref, y2_ref, degp_ref, b2_ref, out_ref):
    dinv = _dinv_block(degp_ref[...])
    p = part_ref[...]
    out_ref[...] = (p[0] + p[1] + y2_ref[...]) * dinv + b2_ref[...]


_tc3 = pl.pallas_call(
    _tc3_body,
    grid=(N_PAD // BN,),
    in_specs=[
        pl.BlockSpec((NC, BN, D_OUT), lambda i: (0, i, 0)),
        pl.BlockSpec((BN, D_OUT), lambda i: (i, 0)),
        pl.BlockSpec((NC, BN, DEG_W), lambda i: (0, i, 0)),
        pl.BlockSpec((1, D_OUT), lambda i: (0, 0)),
    ],
    out_specs=pl.BlockSpec((BN, D_OUT), lambda i: (i, 0)),
    out_shape=jax.ShapeDtypeStruct((N_PAD, D_OUT), jnp.float32),
)


def kernel(x, edge_index, W1, b1, W2, b2):
    pad_e = E_PAD - E
    src3 = jnp.concatenate(
        [edge_index[0], jnp.full((pad_e,), N, jnp.int32)]
    ).reshape(TOTAL_CHUNKS, CHUNK)
    dst3 = jnp.concatenate(
        [edge_index[1], jnp.full((pad_e,), N, jnp.int32)]
    ).reshape(TOTAL_CHUNKS, CHUNK)
    x_pad = jnp.zeros((N_PAD, D_IN), jnp.float32).at[:N].set(x)

    ones_w = jnp.ones((CHUNK, DEG_W), jnp.float32)

    degp = _deg_kernel(dst3, ones_w)
    y1 = _tc1(x_pad, W1, degp)
    part1 = _edge_kernel_h(y1, src3, dst3)
    y2 = _tc2(part1, y1, degp, b1.reshape(1, D_HID), W2)
    part2 = _edge_kernel_o(y2, src3, dst3)
    out = _tc3(part2, y2, degp, b2.reshape(1, D_OUT))
    return out[:N]
